# Initial kernel scaffold; baseline (speedup 1.0000x reference)
import jax
import jax.numpy as jnp
from jax.experimental import pallas as pl


def kernel(node_features, edge_index, edge_attr, msg_w1, msg_b1, msg_bn_g, msg_bn_b, msg_w2, msg_b2, upd_w, upd_b, ln_g, ln_b):
    raise NotImplementedError("write your pallas kernel here")


# speedup vs baseline: 1.8585x; 1.8585x over previous
import functools

import jax
import jax.numpy as jnp
from jax import lax
from jax.experimental import pallas as pl
from jax.experimental.pallas import tpu as pltpu
from jax.experimental.pallas import tpu_sc as plsc

N = 10000
E = 320000
D = 128
ED = 16
H = 256
L = 3

NC = 2
NS = 16
HC = H // NC
EPT = E // NS
CH = 200
NCH = EPT // CH
NPAD = 10240
NPT = NPAD // NS
NZ = 128

_mesh = plsc.VectorSubcoreMesh(core_axis_name="c", subcore_axis_name="s")


@functools.partial(
    pl.kernel,
    mesh=_mesh,
    out_type=(
        jax.ShapeDtypeStruct((NC, E, HC), jnp.float32),
        jax.ShapeDtypeStruct((NC, NS, 2, HC), jnp.float32),
    ),
    scratch_types=(
        pltpu.VMEM((CH,), jnp.int32),
        pltpu.VMEM((CH,), jnp.int32),
        pltpu.VMEM((CH, HC), jnp.float32),
        pltpu.VMEM((CH, HC), jnp.float32),
        pltpu.VMEM((CH, HC), jnp.float32),
        pltpu.VMEM((2, HC), jnp.float32),
        pltpu.SemaphoreType.DMA,
        pltpu.SemaphoreType.DMA,
    ),
)
def _p1(dst2, src2, a_t, b_t, c_t, mpre, stats, di, si, ab, bb, cb, st, sem_a, sem_b):
    cid = lax.axis_index("c")
    sid = lax.axis_index("s")
    zero = jnp.zeros((16,), jnp.float32)

    def chunk(k, accs):
        base = sid * EPT + k * CH
        ibase = cid * E + base
        pltpu.sync_copy(dst2.at[pl.ds(ibase, CH)], di)
        pltpu.sync_copy(src2.at[pl.ds(ibase, CH)], si)
        ga = pltpu.async_copy(a_t.at[di], ab, sem_a)
        gb = pltpu.async_copy(b_t.at[si], bb, sem_b)
        pltpu.sync_copy(c_t.at[cid, pl.ds(base, CH)], cb)
        ga.wait()
        gb.wait()

        def row(r, rc):
            vs = list(rc)
            for j in range(HC // 16):
                sl = pl.ds(j * 16, 16)
                v = ab[r, sl] + bb[r, sl] + cb[r, sl]
                cb[r, sl] = v
                vs[j] = vs[j] + v
                vs[j + 8] = vs[j + 8] + v * v
            return tuple(vs)

        accs = lax.fori_loop(0, CH, row, accs)
        pltpu.sync_copy(cb, mpre.at[cid, pl.ds(base, CH)])
        return accs

    accs = lax.fori_loop(0, NCH, chunk, tuple(zero for _ in range(16)))
    for j in range(HC // 16):
        st[0, pl.ds(j * 16, 16)] = accs[j]
        st[1, pl.ds(j * 16, 16)] = accs[j + 8]
    pltpu.sync_copy(st, stats.at[cid, sid])


@functools.partial(
    pl.kernel,
    mesh=_mesh,
    out_type=jax.ShapeDtypeStruct((NC, NPAD, HC), jnp.float32),
    scratch_types=(
        pltpu.VMEM((CH,), jnp.int32),
        pltpu.VMEM((CH, HC), jnp.float32),
        pltpu.VMEM((2, HC), jnp.float32),
        pltpu.VMEM((NZ, HC), jnp.float32),
        pltpu.VMEM_SHARED((NPAD, HC), jnp.float32),
        pltpu.SemaphoreType.DMA,
    ),
)
def _p2(dst1, mpre, ss, s_out, di, vb, ssb, zb, s_sh, sem):
    cid = lax.axis_index("c")
    sid = lax.axis_index("s")
    zero = jnp.zeros((16,), jnp.float32)

    def zrow(r, _):
        for j in range(HC // 16):
            zb[r, pl.ds(j * 16, 16)] = zero
        return 0

    lax.fori_loop(0, NZ, zrow, 0)
    nbase = sid * NPT
    for z in range(NPT // NZ):
        pltpu.sync_copy(zb, s_sh.at[pl.ds(nbase + z * NZ, NZ)])
    plsc.subcore_barrier()

    pltpu.sync_copy(ss.at[cid], ssb)
    sc = [ssb[0, pl.ds(j * 16, 16)] for j in range(HC // 16)]
    sh = [ssb[1, pl.ds(j * 16, 16)] for j in range(HC // 16)]

    def chunk(k, _):
        base = sid * EPT + k * CH
        pltpu.sync_copy(dst1.at[pl.ds(base, CH)], di)
        pltpu.sync_copy(mpre.at[cid, pl.ds(base, CH)], vb)

        def row(r, _):
            for j in range(HC // 16):
                sl = pl.ds(j * 16, 16)
                vb[r, sl] = jnp.maximum(vb[r, sl] * sc[j] + sh[j], 0.0)
            return 0

        lax.fori_loop(0, CH, row, 0)
        pltpu.sync_copy(vb, s_sh.at[di], add=True)
        return 0

    lax.fori_loop(0, NCH, chunk, 0)
    plsc.subcore_barrier()
    for z in range(NPT // NZ):
        pltpu.sync_copy(s_sh.at[pl.ds(nbase + z * NZ, NZ)], zb)
        pltpu.sync_copy(zb, s_out.at[cid, pl.ds(nbase + z * NZ, NZ)])


@functools.partial(
    pl.kernel,
    mesh=_mesh,
    out_type=jax.ShapeDtypeStruct((NC, NPAD, HC), jnp.float32),
    scratch_types=(
        pltpu.VMEM((CH,), jnp.int32),
        pltpu.VMEM((CH, HC), jnp.float32),
        pltpu.VMEM((NZ, HC), jnp.float32),
        pltpu.VMEM_SHARED((NPAD, HC), jnp.float32),
    ),
)
def _cnt(dst1, c_out, di, ob, zb, c_sh):
    cid = lax.axis_index("c")
    sid = lax.axis_index("s")
    zero = jnp.zeros((16,), jnp.float32)
    one = jnp.full((16,), 1.0, jnp.float32)

    def fillz(r, _):
        for j in range(HC // 16):
            zb[r, pl.ds(j * 16, 16)] = zero
        return 0

    lax.fori_loop(0, NZ, fillz, 0)
    nbase = sid * NPT
    for z in range(NPT // NZ):
        pltpu.sync_copy(zb, c_sh.at[pl.ds(nbase + z * NZ, NZ)])
    plsc.subcore_barrier()

    def fillo(r, _):
        for j in range(HC // 16):
            ob[r, pl.ds(j * 16, 16)] = one
        return 0

    lax.fori_loop(0, CH, fillo, 0)

    def chunk(k, _):
        base = sid * EPT + k * CH
        pltpu.sync_copy(dst1.at[pl.ds(base, CH)], di)
        pltpu.sync_copy(ob, c_sh.at[di], add=True)
        return 0

    lax.fori_loop(0, NCH, chunk, 0)
    plsc.subcore_barrier()
    for z in range(NPT // NZ):
        pltpu.sync_copy(c_sh.at[pl.ds(nbase + z * NZ, NZ)], zb)
        pltpu.sync_copy(zb, c_out.at[cid, pl.ds(nbase + z * NZ, NZ)])


def kernel(node_features, edge_index, edge_attr, msg_w1, msg_b1, msg_bn_g,
           msg_bn_b, msg_w2, msg_b2, upd_w, upd_b, ln_g, ln_b):
    src = edge_index[0]
    dst = edge_index[1]
    dst2 = jnp.concatenate([dst, dst + N])
    src2 = jnp.concatenate([src, src + N])

    cnt = _cnt(dst)[0, :N, 0]
    inv = 1.0 / jnp.maximum(cnt, 1.0)
    has = (cnt > 0.0).astype(jnp.float32)

    h = node_features
    for l in range(L):
        A = h @ msg_w1[l][:D]
        B = h @ msg_w1[l][D:2 * D]
        C = edge_attr @ msg_w1[l][2 * D:] + msg_b1[l]
        A2 = jnp.concatenate([A[:, :HC], A[:, HC:]], axis=0)
        B2 = jnp.concatenate([B[:, :HC], B[:, HC:]], axis=0)
        C2 = jnp.stack([C[:, :HC], C[:, HC:]])

        mpre, stats_p = _p1(dst2, src2, A2, B2, C2)
        stats = stats_p.sum(axis=1)
        s1 = jnp.concatenate([stats[0, 0], stats[1, 0]])
        s2 = jnp.concatenate([stats[0, 1], stats[1, 1]])
        mu = s1 / E
        var = s2 / E - mu * mu
        scale = msg_bn_g[l] * lax.rsqrt(var + 1e-5)
        shift = msg_bn_b[l] - mu * scale
        ss = jnp.stack([jnp.stack([scale[:HC], shift[:HC]]),
                        jnp.stack([scale[HC:], shift[HC:]])])

        S2 = _p2(dst, mpre, ss)
        aggm = jnp.concatenate([S2[0, :N], S2[1, :N]], axis=1) * inv[:, None]
        agg = aggm @ msg_w2[l] + msg_b2[l] * has[:, None]

        u = h @ upd_w[l][:D] + agg @ upd_w[l][D:] + upd_b[l]
        h = h + u
        mu2 = h.mean(axis=-1, keepdims=True)
        var2 = h.var(axis=-1, keepdims=True)
        h = (h - mu2) * lax.rsqrt(var2 + 1e-5) * ln_g[l] + ln_b[l]
    return h
